# Initial kernel scaffold; baseline (speedup 1.0000x reference)
import functools

import jax
import jax.numpy as jnp
from jax import lax
from jax.experimental import pallas as pl
from jax.experimental.pallas import tpu as pltpu
from jax.experimental.pallas import tpu_sc as plsc

N_NODES = 10000
N_EDGES = 320000
D_FEAT = 128
POS_DIM = 4

D_PAD = 144
CHUNK = 128
NC = 2
NS = 16
NW = NC * NS
PER_W = -(-N_EDGES // (NW * CHUNK)) * CHUNK
E_PAD = PER_W * NW
N_CHUNKS = PER_W // CHUNK
ROWS_PER_TILE = 640
ACC_ROWS = ROWS_PER_TILE * NS
PAD_ROW = N_NODES
XP_ROWS = 10048

_mesh = plsc.VectorSubcoreMesh(core_axis_name="c", subcore_axis_name="s")


@functools.partial(
    pl.kernel,
    out_type=jax.ShapeDtypeStruct((NC, ACC_ROWS, D_PAD), jnp.float32),
    mesh=_mesh,
    scratch_types=[
        pltpu.VMEM((CHUNK,), jnp.int32),
        pltpu.VMEM((CHUNK,), jnp.int32),
        pltpu.VMEM((CHUNK, D_PAD), jnp.float32),
        pltpu.VMEM_SHARED((ACC_ROWS, D_PAD), jnp.float32),
        pltpu.SemaphoreType.DMA,
    ],
)
def _sc_scatter_accum(xp_hbm, src_hbm, dst_hbm, z_hbm, out_hbm,
                      sidx, didx, rows, acc, sem):
    c = lax.axis_index("c")
    s = lax.axis_index("s")
    wid = s * NC + c

    pltpu.sync_copy(z_hbm, rows)
    for b in range(ROWS_PER_TILE // CHUNK):
        pltpu.sync_copy(
            rows, acc.at[pl.ds(s * ROWS_PER_TILE + b * CHUNK, CHUNK)])
    plsc.subcore_barrier()

    base = wid * PER_W

    def body(g, carry):
        off = base + g * CHUNK
        pltpu.sync_copy(src_hbm.at[pl.ds(off, CHUNK)], sidx)
        pltpu.sync_copy(dst_hbm.at[pl.ds(off, CHUNK)], didx)
        pltpu.async_copy(xp_hbm.at[sidx], rows, sem).wait()
        pltpu.sync_copy(rows, acc.at[didx], add=True)
        return carry

    lax.fori_loop(0, N_CHUNKS, body, 0)
    plsc.subcore_barrier()

    pltpu.sync_copy(
        acc.at[pl.ds(s * ROWS_PER_TILE, ROWS_PER_TILE)],
        out_hbm.at[c, pl.ds(s * ROWS_PER_TILE, ROWS_PER_TILE)],
    )


_R = 500


def _combine_body(part_ref, x_ref, pos_ref, out_ref):
    p = part_ref[0] + part_ref[1]
    deg = p[:, D_FEAT + POS_DIM:D_FEAT + POS_DIM + 1]
    outx = p[:, :D_FEAT] + x_ref[...]
    outp = p[:, D_FEAT:D_FEAT + POS_DIM] - deg * pos_ref[...]
    out_ref[...] = jnp.concatenate([outx, outp], axis=1)


_combine = pl.pallas_call(
    _combine_body,
    grid=(N_NODES // _R,),
    in_specs=[
        pl.BlockSpec((NC, _R, D_PAD), lambda i: (0, i, 0)),
        pl.BlockSpec((_R, D_FEAT), lambda i: (i, 0)),
        pl.BlockSpec((_R, POS_DIM), lambda i: (i, 0)),
    ],
    out_specs=pl.BlockSpec((_R, D_FEAT + POS_DIM), lambda i: (i, 0)),
    out_shape=jax.ShapeDtypeStruct((N_NODES, D_FEAT + POS_DIM), jnp.float32),
)


def kernel(x, pos, edge_index):
    ei = edge_index.astype(jnp.int32)
    npad = E_PAD - N_EDGES
    src = jnp.concatenate([ei[0], jnp.full((npad,), PAD_ROW, jnp.int32)])
    dst = jnp.concatenate([ei[1], jnp.full((npad,), PAD_ROW, jnp.int32)])
    xp = jnp.concatenate(
        [x, pos, jnp.ones((N_NODES, 1), jnp.float32)], axis=1)
    xp = jnp.pad(xp, ((0, XP_ROWS - N_NODES), (0, D_PAD - (D_FEAT + POS_DIM + 1))))
    zeros_chunk = jnp.zeros((CHUNK, D_PAD), jnp.float32)
    part = _sc_scatter_accum(xp, src, dst, zeros_chunk)
    return _combine(part, x, pos)


# speedup vs baseline: 8.6652x; 8.6652x over previous
import functools

import jax
import jax.numpy as jnp
from jax import lax
from jax.experimental import pallas as pl
from jax.experimental.pallas import tpu as pltpu
from jax.experimental.pallas import tpu_sc as plsc

N_NODES = 10000
N_EDGES = 320000
D_FEAT = 128
POS_DIM = 4

D_PAD = 144
CHUNK = 128
NC = 2
NS = 16
NW = NC * NS
PER_W = -(-N_EDGES // (NW * CHUNK)) * CHUNK
E_PAD = PER_W * NW
N_CHUNKS = PER_W // CHUNK
ROWS_PER_TILE = 640
ACC_ROWS = ROWS_PER_TILE * NS
PAD_ROW = N_NODES
XP_ROWS = 10048

_mesh = plsc.VectorSubcoreMesh(core_axis_name="c", subcore_axis_name="s")


@functools.partial(
    pl.kernel,
    out_type=jax.ShapeDtypeStruct((NC, ACC_ROWS, D_PAD), jnp.float32),
    mesh=_mesh,
    scratch_types=[
        pltpu.VMEM((CHUNK,), jnp.int32),
        pltpu.VMEM((CHUNK,), jnp.int32),
        pltpu.VMEM((CHUNK, D_PAD), jnp.float32),
        pltpu.VMEM_SHARED((ACC_ROWS, D_PAD), jnp.float32),
        pltpu.SemaphoreType.DMA,
    ],
    compiler_params=pltpu.CompilerParams(use_tc_tiling_on_sc=False),
)
def _sc_scatter_accum(xp_hbm, src_hbm, dst_hbm, z_hbm, out_hbm,
                      sidx, didx, rows, acc, sem):
    c = lax.axis_index("c")
    s = lax.axis_index("s")
    wid = s * NC + c

    pltpu.sync_copy(z_hbm, rows)
    for b in range(ROWS_PER_TILE // CHUNK):
        pltpu.sync_copy(
            rows, acc.at[pl.ds(s * ROWS_PER_TILE + b * CHUNK, CHUNK)])
    plsc.subcore_barrier()

    base = wid * PER_W

    def body(g, carry):
        off = base + g * CHUNK
        pltpu.sync_copy(src_hbm.at[pl.ds(off, CHUNK)], sidx)
        pltpu.sync_copy(dst_hbm.at[pl.ds(off, CHUNK)], didx)
        pltpu.async_copy(xp_hbm.at[sidx], rows, sem).wait()
        pltpu.sync_copy(rows, acc.at[didx], add=True)
        return carry

    lax.fori_loop(0, N_CHUNKS, body, 0)
    plsc.subcore_barrier()

    pltpu.sync_copy(
        acc.at[pl.ds(s * ROWS_PER_TILE, ROWS_PER_TILE)],
        out_hbm.at[c, pl.ds(s * ROWS_PER_TILE, ROWS_PER_TILE)],
    )


_R = 400


def _combine_body(part_ref, x_ref, pos_ref, out_ref):
    p = part_ref[0] + part_ref[1]
    deg = p[:, D_FEAT + POS_DIM:D_FEAT + POS_DIM + 1]
    outx = p[:, :D_FEAT] + x_ref[...]
    outp = p[:, D_FEAT:D_FEAT + POS_DIM] - deg * pos_ref[...]
    out_ref[...] = jnp.concatenate([outx, outp], axis=1)


_combine = pl.pallas_call(
    _combine_body,
    grid=(N_NODES // _R,),
    in_specs=[
        pl.BlockSpec((NC, _R, D_PAD), lambda i: (0, i, 0)),
        pl.BlockSpec((_R, D_FEAT), lambda i: (i, 0)),
        pl.BlockSpec((_R, POS_DIM), lambda i: (i, 0)),
    ],
    out_specs=pl.BlockSpec((_R, D_FEAT + POS_DIM), lambda i: (i, 0)),
    out_shape=jax.ShapeDtypeStruct((N_NODES, D_FEAT + POS_DIM), jnp.float32),
)


def kernel(x, pos, edge_index):
    ei = edge_index.astype(jnp.int32)
    npad = E_PAD - N_EDGES
    src = jnp.concatenate([ei[0], jnp.full((npad,), PAD_ROW, jnp.int32)])
    dst = jnp.concatenate([ei[1], jnp.full((npad,), PAD_ROW, jnp.int32)])
    xp = jnp.concatenate(
        [x, pos, jnp.ones((N_NODES, 1), jnp.float32)], axis=1)
    xp = jnp.pad(xp, ((0, XP_ROWS - N_NODES), (0, D_PAD - (D_FEAT + POS_DIM + 1))))
    zeros_chunk = jnp.zeros((CHUNK, D_PAD), jnp.float32)
    part = _sc_scatter_accum(xp, src, dst, zeros_chunk)
    return _combine(part, x, pos)
